# baseline (speedup 1.0000x reference)
import jax
import jax.numpy as jnp
from jax import lax
from jax.experimental import pallas as pl
from jax.experimental.pallas import tpu as pltpu

N_WORD = 100000
N_DIM = 64
CONTEXT = 4
NCTX = 2 * CONTEXT
HIDDEN = 128
IN_DIM = NCTX * N_DIM

NB = 2
R = N_WORD // NB


def _body(x_ref, *refs):
    emb_refs = refs[:NCTX]
    w1_ref, b1_ref, w2_ref, b2_ref, out_ref, hrel_ref, sc_ref = refs[NCTX:]
    j = pl.program_id(0)

    @pl.when(j == 0)
    def _():
        h1 = b1_ref[...].reshape(1, HIDDEN)
        lane = lax.broadcasted_iota(jnp.int32, (1, 128), 1)
        for i in range(NCTX):
            onehot = (lane == x_ref[i] % 128).astype(jnp.float32)
            col = lax.dot_general(
                onehot, emb_refs[i][...], (((1,), (1,)), ((), ())),
                preferred_element_type=jnp.float32)
            h1 = h1 + lax.dot_general(
                col, w1_ref[:, i * N_DIM:(i + 1) * N_DIM],
                (((1,), (1,)), ((), ())),
                preferred_element_type=jnp.float32)
        hrel_ref[...] = jnp.maximum(h1, 0.0)

    @pl.when(j < NB)
    def _():
        logits = lax.dot_general(
            hrel_ref[...], w2_ref[...], (((1,), (1,)), ((), ())),
            preferred_element_type=jnp.float32)
        sc_ref[pl.ds(j, 1), :] = logits

    @pl.when(j == NB)
    def _():
        for jj in range(NB):
            sc_ref[jj:jj + 1, :] = (
                sc_ref[jj:jj + 1, :]
                + b2_ref[pl.ds(jj * R, R)].reshape(1, R))
        h2 = sc_ref[...]
        m = jnp.max(h2)
        norm = m + jnp.log(jnp.sum(jnp.exp(h2 - m)))
        res = h2 - norm
        for jj in range(NB):
            out_ref[0:1, jj * R:(jj + 1) * R] = res[jj:jj + 1, :]


@jax.jit
def kernel(x, emb, w1, b1, w2, b2):
    xi = x.astype(jnp.int32)
    embt = emb.T
    emb_spec = [
        pl.BlockSpec((N_DIM, 128), lambda j, xr, i=i: (0, xr[i] // 128))
        for i in range(NCTX)
    ]
    return pl.pallas_call(
        _body,
        grid_spec=pltpu.PrefetchScalarGridSpec(
            num_scalar_prefetch=1,
            grid=(NB + 1,),
            in_specs=emb_spec + [
                pl.BlockSpec((HIDDEN, IN_DIM), lambda j, xr: (0, 0)),
                pl.BlockSpec(memory_space=pltpu.VMEM),
                pl.BlockSpec((R, HIDDEN), lambda j, xr: (jnp.minimum(j, NB - 1), 0)),
                pl.BlockSpec(memory_space=pltpu.VMEM),
            ],
            out_specs=pl.BlockSpec((1, N_WORD), lambda j, xr: (0, 0)),
            scratch_shapes=[
                pltpu.VMEM((1, HIDDEN), jnp.float32),
                pltpu.VMEM((NB, R), jnp.float32),
            ],
        ),
        out_shape=jax.ShapeDtypeStruct((1, N_WORD), jnp.float32),
        compiler_params=pltpu.CompilerParams(
            dimension_semantics=("arbitrary",)),
    )(xi, *([embt] * NCTX), w1, b1, w2, b2)


# speedup vs baseline: 1.0982x; 1.0982x over previous
import jax
import jax.numpy as jnp
from jax import lax
from jax.experimental import pallas as pl
from jax.experimental.pallas import tpu as pltpu

N_WORD = 100000
N_DIM = 64
CONTEXT = 4
NCTX = 2 * CONTEXT
HIDDEN = 128
IN_DIM = NCTX * N_DIM

NB = 4
R = N_WORD // NB


def _body(x_ref, *refs):
    emb_refs = refs[:NCTX]
    w1_ref, b1_ref, w2_ref, b2_ref, out_ref, hrel_ref, sc_ref = refs[NCTX:]
    j = pl.program_id(0)

    @pl.when(j == 0)
    def _():
        h1 = b1_ref[...].reshape(1, HIDDEN)
        lane = lax.broadcasted_iota(jnp.int32, (1, 128), 1)
        for i in range(NCTX):
            onehot = (lane == x_ref[i] % 128).astype(jnp.float32)
            col = lax.dot_general(
                onehot, emb_refs[i][...], (((1,), (1,)), ((), ())),
                preferred_element_type=jnp.float32)
            h1 = h1 + lax.dot_general(
                col, w1_ref[:, i * N_DIM:(i + 1) * N_DIM],
                (((1,), (1,)), ((), ())),
                preferred_element_type=jnp.float32)
        hrel_ref[...] = jnp.maximum(h1, 0.0)

    @pl.when(j < NB)
    def _():
        logits = lax.dot_general(
            hrel_ref[...], w2_ref[...], (((1,), (1,)), ((), ())),
            preferred_element_type=jnp.float32,
            precision=lax.Precision.DEFAULT)
        sc_ref[pl.ds(j, 1), :] = logits

    @pl.when(j == NB)
    def _():
        for jj in range(NB):
            sc_ref[jj:jj + 1, :] = (
                sc_ref[jj:jj + 1, :]
                + b2_ref[pl.ds(jj * R, R)].reshape(1, R))
        h2 = sc_ref[...]
        m = jnp.max(h2)
        norm = m + jnp.log(jnp.sum(jnp.exp(h2 - m)))
        res = h2 - norm
        for jj in range(NB):
            out_ref[0:1, jj * R:(jj + 1) * R] = res[jj:jj + 1, :]


@jax.jit
def kernel(x, emb, w1, b1, w2, b2):
    xi = x.astype(jnp.int32)
    embt = emb.T
    emb_spec = [
        pl.BlockSpec((N_DIM, 128), lambda j, xr, i=i: (0, xr[i] // 128))
        for i in range(NCTX)
    ]
    return pl.pallas_call(
        _body,
        grid_spec=pltpu.PrefetchScalarGridSpec(
            num_scalar_prefetch=1,
            grid=(NB + 1,),
            in_specs=emb_spec + [
                pl.BlockSpec((HIDDEN, IN_DIM), lambda j, xr: (0, 0)),
                pl.BlockSpec(memory_space=pltpu.VMEM),
                pl.BlockSpec((R, HIDDEN), lambda j, xr: (jnp.minimum(j, NB - 1), 0)),
                pl.BlockSpec(memory_space=pltpu.VMEM),
            ],
            out_specs=pl.BlockSpec((1, N_WORD), lambda j, xr: (0, 0)),
            scratch_shapes=[
                pltpu.VMEM((1, HIDDEN), jnp.float32),
                pltpu.VMEM((NB, R), jnp.float32),
            ],
        ),
        out_shape=jax.ShapeDtypeStruct((1, N_WORD), jnp.float32),
        compiler_params=pltpu.CompilerParams(
            dimension_semantics=("arbitrary",)),
    )(xi, *([embt] * NCTX), w1, b1, w2, b2)
